# baseline (speedup 1.0000x reference)
import functools
import math

import jax
import jax.numpy as jnp
import numpy as np
from jax import lax
from jax.experimental import pallas as pl
from jax.experimental.pallas import tpu as pltpu
from jax.experimental.pallas import tpu_sc as plsc

_DIM = 1024
_HALF = _DIM // 2
_BASE = 10000.0
_DIST_SCALE = 16.0
_INV_SQRT_DIM = 1.0 / math.sqrt(_DIM)


def _fit_turn_polys():
    v = np.linspace(-0.5, 0.5, 40001)
    w = v * v
    cosy = np.cos(2 * np.pi * v)
    siny = np.where(v == 0, 2 * np.pi, np.sin(2 * np.pi * v) / np.where(v == 0, 1, v))
    pc = np.polynomial.Polynomial.fit(w, cosy, 3).convert().coef
    ps = np.polynomial.Polynomial.fit(w, siny, 3).convert().coef
    return (tuple(float(c) * _INV_SQRT_DIM for c in pc),
            tuple(float(c) * _INV_SQRT_DIM for c in ps))


_COS_COEF, _SIN_COEF = _fit_turn_polys()


def _freq_reduction_consts():
    inv_freq32 = (
        1.0 / (_BASE ** (np.arange(0, _DIM, 2).astype(np.float32) / np.float32(_DIM)))
    ).astype(np.float32)
    f = np.mod(_DIST_SCALE * inv_freq32.astype(np.float64) / (2 * np.pi), 1.0)
    return f.astype(np.float32).reshape(1, _HALF)


_F_TURNS = _freq_reduction_consts()


def _sc_gather(weight, idx_flat, n_tokens):
    info = plsc.get_sparse_core_info()
    nw = info.num_cores * info.num_subcores
    b_per_w = n_tokens // nw
    chunk = 16
    nbuf = 4
    n_chunks = b_per_w // chunk
    mesh = plsc.VectorSubcoreMesh(core_axis_name="c", subcore_axis_name="s")

    @functools.partial(
        pl.kernel,
        mesh=mesh,
        out_type=jax.ShapeDtypeStruct((n_tokens, _DIM), jnp.float32),
        scratch_types=[
            pltpu.VMEM((b_per_w,), jnp.int32),
            *[pltpu.VMEM((chunk, _DIM), jnp.float32) for _ in range(nbuf)],
            *[pltpu.SemaphoreType.DMA for _ in range(2 * nbuf)],
        ],
    )
    def gather_kernel(table_hbm, idx_hbm, out_hbm, idx_v, *bufs_and_sems):
        wid = lax.axis_index("s") * info.num_cores + lax.axis_index("c")
        base = wid * b_per_w
        pltpu.sync_copy(idx_hbm.at[pl.ds(base, b_per_w)], idx_v)

        bufs = bufs_and_sems[:nbuf]
        gsems = bufs_and_sems[nbuf:2 * nbuf]
        ssems = bufs_and_sems[2 * nbuf:]

        def gather_start(j):
            pltpu.async_copy(
                table_hbm.at[idx_v.at[pl.ds(j * chunk, chunk)]],
                bufs[j % nbuf], gsems[j % nbuf],
            )

        def gather_wait(j):
            pltpu.make_async_copy(
                table_hbm.at[idx_v.at[pl.ds(j * chunk, chunk)]],
                bufs[j % nbuf], gsems[j % nbuf],
            ).wait()

        def scatter_start(j):
            pltpu.async_copy(
                bufs[j % nbuf], out_hbm.at[pl.ds(base + j * chunk, chunk)],
                ssems[j % nbuf],
            )

        def scatter_wait(j):
            pltpu.make_async_copy(
                bufs[j % nbuf], out_hbm.at[pl.ds(base + j * chunk, chunk)],
                ssems[j % nbuf],
            ).wait()

        for j in range(nbuf - 1):
            gather_start(j)
        for j in range(n_chunks):
            gather_wait(j)
            if j + nbuf - 1 < n_chunks:
                if j >= 1:
                    scatter_wait(j - 1)
                gather_start(j + nbuf - 1)
            scatter_start(j)
        for j in range(n_chunks - nbuf, n_chunks):
            scatter_wait(j)

    return gather_kernel(weight, idx_flat)


def _horner(w, coef):
    acc = jnp.full_like(w, coef[-1])
    for c in coef[-2::-1]:
        acc = acc * w + c
    return acc


def _rotary_body(f_ref, n_ref, x_ref, o_ref):
    n = n_ref[...]
    f = f_ref[...]
    z = n * f
    v = z - jnp.round(z)
    w = v * v
    c = _horner(w, _COS_COEF)
    s = v * _horner(w, _SIN_COEF)
    x1 = x_ref[:, :_HALF]
    x2 = x_ref[:, _HALF:]
    o_ref[:, :_HALF] = x1 * c - x2 * s
    o_ref[:, _HALF:] = x2 * c + x1 * s


def _rotary_tc(embeds, nsub):
    n_tokens = embeds.shape[0]
    t = 512
    grid = (n_tokens // t,)
    return pl.pallas_call(
        _rotary_body,
        grid=grid,
        in_specs=[
            pl.BlockSpec((1, _HALF), lambda i: (0, 0)),
            pl.BlockSpec((t, 1), lambda i: (i, 0)),
            pl.BlockSpec((t, _DIM), lambda i: (i, 0)),
        ],
        out_specs=pl.BlockSpec((t, _DIM), lambda i: (i, 0)),
        out_shape=jax.ShapeDtypeStruct((n_tokens, _DIM), jnp.float32),
    )(jnp.asarray(_F_TURNS), nsub, embeds)


def kernel(ids, ids_sub, weight):
    b, s = ids.shape
    n = b * s
    idx = ids.reshape(n)
    nsub = ids_sub.astype(jnp.float32).reshape(n, 1)
    embeds = _sc_gather(weight, idx, n)
    out = _rotary_tc(embeds, nsub)
    return out.reshape(b, s, _DIM)


# speedup vs baseline: 1.0087x; 1.0087x over previous
import functools
import math

import jax
import jax.numpy as jnp
import numpy as np
from jax import lax
from jax.experimental import pallas as pl
from jax.experimental.pallas import tpu as pltpu
from jax.experimental.pallas import tpu_sc as plsc

_DIM = 1024
_HALF = _DIM // 2
_BASE = 10000.0
_DIST_SCALE = 16.0
_INV_SQRT_DIM = 1.0 / math.sqrt(_DIM)


def _fit_turn_polys():
    v = np.linspace(-0.5, 0.5, 40001)
    w = v * v
    cosy = np.cos(2 * np.pi * v)
    siny = np.where(v == 0, 2 * np.pi, np.sin(2 * np.pi * v) / np.where(v == 0, 1, v))
    pc = np.polynomial.Polynomial.fit(w, cosy, 3).convert().coef
    ps = np.polynomial.Polynomial.fit(w, siny, 3).convert().coef
    return (tuple(float(c) * _INV_SQRT_DIM for c in pc),
            tuple(float(c) * _INV_SQRT_DIM for c in ps))


_COS_COEF, _SIN_COEF = _fit_turn_polys()


def _freq_reduction_consts():
    inv_freq32 = (
        1.0 / (_BASE ** (np.arange(0, _DIM, 2).astype(np.float32) / np.float32(_DIM)))
    ).astype(np.float32)
    f = np.mod(_DIST_SCALE * inv_freq32.astype(np.float64) / (2 * np.pi), 1.0)
    return f.astype(np.float32).reshape(1, _HALF)


_F_TURNS = _freq_reduction_consts()


def _sc_gather(weight, idx_flat, n_tokens):
    info = plsc.get_sparse_core_info()
    nw = info.num_cores * info.num_subcores
    b_per_w = n_tokens // nw
    chunk = 32
    nbuf = 3
    n_chunks = b_per_w // chunk
    mesh = plsc.VectorSubcoreMesh(core_axis_name="c", subcore_axis_name="s")

    @functools.partial(
        pl.kernel,
        mesh=mesh,
        out_type=jax.ShapeDtypeStruct((n_tokens, _DIM), jnp.float32),
        scratch_types=[
            pltpu.VMEM((b_per_w,), jnp.int32),
            *[pltpu.VMEM((chunk, _DIM), jnp.float32) for _ in range(nbuf)],
            *[pltpu.SemaphoreType.DMA for _ in range(2 * nbuf)],
        ],
    )
    def gather_kernel(table_hbm, idx_hbm, out_hbm, idx_v, *bufs_and_sems):
        wid = lax.axis_index("s") * info.num_cores + lax.axis_index("c")
        base = wid * b_per_w
        pltpu.sync_copy(idx_hbm.at[pl.ds(base, b_per_w)], idx_v)

        bufs = bufs_and_sems[:nbuf]
        gsems = bufs_and_sems[nbuf:2 * nbuf]
        ssems = bufs_and_sems[2 * nbuf:]

        def gather_start(j):
            pltpu.async_copy(
                table_hbm.at[idx_v.at[pl.ds(j * chunk, chunk)]],
                bufs[j % nbuf], gsems[j % nbuf],
            )

        def gather_wait(j):
            pltpu.make_async_copy(
                table_hbm.at[idx_v.at[pl.ds(j * chunk, chunk)]],
                bufs[j % nbuf], gsems[j % nbuf],
            ).wait()

        def scatter_start(j):
            pltpu.async_copy(
                bufs[j % nbuf], out_hbm.at[pl.ds(base + j * chunk, chunk)],
                ssems[j % nbuf],
            )

        def scatter_wait(j):
            pltpu.make_async_copy(
                bufs[j % nbuf], out_hbm.at[pl.ds(base + j * chunk, chunk)],
                ssems[j % nbuf],
            ).wait()

        for j in range(nbuf - 1):
            gather_start(j)
        for j in range(n_chunks):
            gather_wait(j)
            if j + nbuf - 1 < n_chunks:
                if j >= 1:
                    scatter_wait(j - 1)
                gather_start(j + nbuf - 1)
            scatter_start(j)
        for j in range(n_chunks - nbuf, n_chunks):
            scatter_wait(j)

    return gather_kernel(weight, idx_flat)


def _horner(w, coef):
    acc = jnp.full_like(w, coef[-1])
    for c in coef[-2::-1]:
        acc = acc * w + c
    return acc


def _rotary_body(f_ref, n_ref, x_ref, o_ref):
    n = n_ref[...]
    f = f_ref[...]
    z = n * f
    v = z - jnp.round(z)
    w = v * v
    c = _horner(w, _COS_COEF)
    s = v * _horner(w, _SIN_COEF)
    x1 = x_ref[:, :_HALF]
    x2 = x_ref[:, _HALF:]
    o_ref[:, :_HALF] = x1 * c - x2 * s
    o_ref[:, _HALF:] = x2 * c + x1 * s


def _rotary_tc(embeds, nsub):
    n_tokens = embeds.shape[0]
    t = 512
    grid = (n_tokens // t,)
    return pl.pallas_call(
        _rotary_body,
        grid=grid,
        in_specs=[
            pl.BlockSpec((1, _HALF), lambda i: (0, 0)),
            pl.BlockSpec((t, 1), lambda i: (i, 0)),
            pl.BlockSpec((t, _DIM), lambda i: (i, 0)),
        ],
        out_specs=pl.BlockSpec((t, _DIM), lambda i: (i, 0)),
        out_shape=jax.ShapeDtypeStruct((n_tokens, _DIM), jnp.float32),
    )(jnp.asarray(_F_TURNS), nsub, embeds)


def kernel(ids, ids_sub, weight):
    b, s = ids.shape
    n = b * s
    idx = ids.reshape(n)
    nsub = ids_sub.astype(jnp.float32).reshape(n, 1)
    embeds = _sc_gather(weight, idx, n)
    out = _rotary_tc(embeds, nsub)
    return out.reshape(b, s, _DIM)
